# baseline (speedup 1.0000x reference)
import functools

import jax
import jax.numpy as jnp
from jax import lax
from jax.experimental import pallas as pl
from jax.experimental.pallas import tpu as pltpu
from jax.experimental.pallas import tpu_sc as plsc

_K = 1024
_D = 64
_BETA = 0.25
_HW = 1024
_B = 16
_N = _B * _HW

_BB = 2

_NC = 2
_NS = 16
_NW = _NC * _NS
_ROWS_PER_W = _N // _NW


def _vq_block(z_ref, w_ref, idx_ref, loss_ref):
    i = pl.program_id(0)
    w = w_ref[...]
    w2 = jnp.sum(w ** 2, axis=1, keepdims=True)

    @pl.when(i == 0)
    def _init():
        loss_ref[...] = jnp.zeros_like(loss_ref)

    w2x = w + w
    for j in range(_BB):
        zd = z_ref[j]
        z2 = jnp.sum(zd ** 2, axis=0, keepdims=True)
        s2 = jax.lax.dot_general(
            w2x, zd, (((1,), (0,)), ((), ())),
            preferred_element_type=jnp.float32)
        d2 = (z2 + w2) - s2
        m = jnp.min(d2, axis=0, keepdims=True)
        iota = jax.lax.broadcasted_iota(jnp.int32, d2.shape, 0)
        cand = jnp.where(d2 == m, iota, _K)
        idx = jnp.min(cand, axis=0, keepdims=True)
        idx_ref[j] = jnp.broadcast_to(idx, (8, _HW))
        loss_ref[...] += jnp.sum(m) * ((1.0 + _BETA) / (_N * _D))


_SC_MESH = plsc.VectorSubcoreMesh(core_axis_name="c", subcore_axis_name="s")


@functools.partial(
    pl.kernel,
    out_type=jax.ShapeDtypeStruct((_N, 128), jnp.float32),
    mesh=_SC_MESH,
    scratch_types=[
        pltpu.VMEM((_ROWS_PER_W,), jnp.int32),
        pltpu.VMEM((_ROWS_PER_W, 128), jnp.float32),
        pltpu.SemaphoreType.DMA,
    ],
)
def _sc_gather(w_hbm, idx_hbm, out_hbm, idx_v, rows_v, sem):
    wid = lax.axis_index("s") * _NC + lax.axis_index("c")
    base = wid * _ROWS_PER_W
    pltpu.sync_copy(idx_hbm.at[pl.ds(base, _ROWS_PER_W)], idx_v)
    pltpu.async_copy(w_hbm.at[idx_v], rows_v, sem).wait()
    pltpu.sync_copy(rows_v, out_hbm.at[pl.ds(base, _ROWS_PER_W)])


def kernel(z, W):
    zr = z.reshape(_B, _D, _HW)
    idx3, loss = pl.pallas_call(
        _vq_block,
        grid=(_B // _BB,),
        in_specs=[
            pl.BlockSpec((_BB, _D, _HW), lambda i: (i, 0, 0)),
            pl.BlockSpec((_K, _D), lambda i: (0, 0)),
        ],
        out_specs=[
            pl.BlockSpec((_BB, 8, _HW), lambda i: (i, 0, 0)),
            pl.BlockSpec((1, 1), lambda i: (0, 0)),
        ],
        out_shape=[
            jax.ShapeDtypeStruct((_B, 8, _HW), jnp.int32),
            jax.ShapeDtypeStruct((1, 1), jnp.float32),
        ],
    )(zr, W)
    idx_flat = idx3[:, 0, :].reshape(-1)
    W128 = jnp.pad(W, ((0, 0), (0, 128 - _D)))
    zq_rows = _sc_gather(W128, idx_flat)
    out = jnp.transpose(zq_rows.reshape(_B, 32, 32, 128),
                        (0, 3, 1, 2))[:, :_D]
    return out, loss[0, 0]
